# baseline (speedup 1.0000x reference)
import jax
import jax.numpy as jnp
from jax.experimental import pallas as pl
from jax.experimental.pallas import tpu as pltpu


def _conv_body(L, C, SP, B, PW):
    SPB = B * SP
    deltas = [(dy - 1) * PW + (dx - 1) for dy in range(3) for dx in range(3)]
    d4a = [deltas[t] for t in (0, 1, 2, 3)]
    d4b = [deltas[t] for t in (5, 6, 7, 8)]

    def body(x_ref, mask_ref, w4a_ref, w4b_ref, wc_ref, bc_ref, o_ref,
             scr_ref, xa_ref, xb_ref):
        CIN8 = x_ref.shape[2]
        scr_ref[CIN8:, :] = jnp.zeros((C - CIN8, SPB), jnp.bfloat16)
        for b in range(B):
            scr_ref[:CIN8, b * SP:(b + 1) * SP] = x_ref[0, b]
        mask = mask_ref[...]
        for l in range(L):
            av = scr_ref[...]
            for j, d in enumerate(d4a):
                xa_ref[j * C:(j + 1) * C, :] = jnp.roll(av, -d, axis=1)
            for j, d in enumerate(d4b):
                xb_ref[j * C:(j + 1) * C, :] = jnp.roll(av, -d, axis=1)
            z = jnp.dot(w4a_ref[l], xa_ref[...],
                        preferred_element_type=jnp.float32)
            z = z + jnp.dot(w4b_ref[l], xb_ref[...],
                            preferred_element_type=jnp.float32)
            z = z + jnp.dot(wc_ref[l], av,
                            preferred_element_type=jnp.float32)
            ab = jnp.maximum(z + bc_ref[l], 0.0).astype(jnp.bfloat16) * mask
            if l + 1 < L:
                scr_ref[...] = ab
            else:
                for b in range(B):
                    o_ref[0, b] = ab[:, b * SP:(b + 1) * SP]
    return body


def _fc_body(C, SP):
    def fc(r_ref, w_ref, b_ref, o_ref):
        acc = None
        for c in range(C):
            p = jax.lax.dot_general(
                r_ref[:, c * SP:(c + 1) * SP], w_ref[c],
                (((1,), (1,)), ((), ())),
                preferred_element_type=jnp.float32)
            acc = p if acc is None else acc + p
        o_ref[...] = acc + b_ref[...]
    return fc


def kernel(x_nchw, wc, bc, wfc, bfc, mask):
    N, cin, H, W = x_nchw.shape
    L = wc.shape[0]
    C = wc.shape[2]
    NC = bfc.shape[1]
    SP = wfc.shape[2]
    PH, PW = H + 2, W + 2
    assert SP >= PH * PW and SP % 128 == 0 and cin <= C
    assert SP - (PH - 1) * PW - (PW - 1) > PW + 1 > 0

    B = 32
    steps = -(-N // B)
    N_pad = steps * B
    SPB = B * SP

    CIN8 = min(C, ((cin + 7) // 8) * 8)
    xp = jnp.pad(x_nchw.astype(jnp.bfloat16),
                 ((0, N_pad - N), (0, CIN8 - cin), (1, 1), (1, 1)))
    xp = xp.reshape(N_pad, CIN8, PH * PW)
    xp = jnp.pad(xp, ((0, 0), (0, 0), (0, SP - PH * PW)))
    xp = xp.reshape(steps, B, CIN8, SP)
    mask_b = jnp.tile(mask, (1, B)).astype(jnp.bfloat16)

    w_all = jnp.transpose(wc, (0, 2, 1, 3))
    w4a = w_all[:, :, [0, 1, 2, 3], :].reshape(L, C, 4 * C).astype(jnp.bfloat16)
    w4b = w_all[:, :, [5, 6, 7, 8], :].reshape(L, C, 4 * C).astype(jnp.bfloat16)
    wcen = w_all[:, :, 4, :].astype(jnp.bfloat16)

    act = pl.pallas_call(
        _conv_body(L, C, SP, B, PW),
        out_shape=jax.ShapeDtypeStruct((steps, B, C, SP), jnp.bfloat16),
        grid=(steps,),
        in_specs=[
            pl.BlockSpec((1, B, CIN8, SP), lambda s: (s, 0, 0, 0)),
            pl.BlockSpec((1, SPB), lambda s: (0, 0)),
            pl.BlockSpec((L, C, 4 * C), lambda s: (0, 0, 0)),
            pl.BlockSpec((L, C, 4 * C), lambda s: (0, 0, 0)),
            pl.BlockSpec((L, C, C), lambda s: (0, 0, 0)),
            pl.BlockSpec((L, C, 1), lambda s: (0, 0, 0)),
        ],
        out_specs=pl.BlockSpec((1, B, C, SP), lambda s: (s, 0, 0, 0)),
        scratch_shapes=[
            pltpu.VMEM((C, SPB), jnp.bfloat16),
            pltpu.VMEM((4 * C, SPB), jnp.bfloat16),
            pltpu.VMEM((4 * C, SPB), jnp.bfloat16),
        ],
        compiler_params=pltpu.CompilerParams(
            dimension_semantics=("arbitrary",)),
    )(xp, mask_b, w4a, w4b, wcen, bc)

    rows = act.reshape(N_pad, C * SP)
    wfcb = wfc.astype(jnp.bfloat16)

    MB = 256 if N_pad % 256 == 0 else B
    fsteps = N_pad // MB
    scores = pl.pallas_call(
        _fc_body(C, SP),
        out_shape=jax.ShapeDtypeStruct((N_pad, NC), jnp.float32),
        grid=(fsteps,),
        in_specs=[
            pl.BlockSpec((MB, C * SP), lambda s: (s, 0)),
            pl.BlockSpec((C, NC, SP), lambda s: (0, 0, 0)),
            pl.BlockSpec((1, NC), lambda s: (0, 0)),
        ],
        out_specs=pl.BlockSpec((MB, NC), lambda s: (s, 0)),
        compiler_params=pltpu.CompilerParams(
            dimension_semantics=("parallel",)),
    )(rows, wfcb, bfc)

    return scores[:N], None, None


# speedup vs baseline: 1.0821x; 1.0821x over previous
import jax
import jax.numpy as jnp
from jax.experimental import pallas as pl
from jax.experimental.pallas import tpu as pltpu


def _conv_body(L, C, SP, B, PW):
    SPB = B * SP
    deltas = [(dy - 1) * PW + (dx - 1) for dy in range(3) for dx in range(3)]
    d4a = [deltas[t] for t in (0, 1, 2, 3)]
    d4b = [deltas[t] for t in (5, 6, 7, 8)]

    def body(x_ref, mask_ref, w4a_ref, w4b_ref, wc_ref, bc_ref,
             w4a0_ref, w4b0_ref, wc0_ref, o_ref, scr_ref, xa_ref, xb_ref):
        CIN8 = x_ref.shape[2]
        K0 = 4 * CIN8
        for b in range(B):
            scr_ref[:CIN8, b * SP:(b + 1) * SP] = x_ref[0, b]
        mask = mask_ref[...]

        def layer(av, xa_k, wa, wb, wcen, bcl, last):
            for j, d in enumerate(d4a):
                xa_ref[j * xa_k:(j + 1) * xa_k, :] = jnp.roll(av, -d, axis=1)
            for j, d in enumerate(d4b):
                xb_ref[j * xa_k:(j + 1) * xa_k, :] = jnp.roll(av, -d, axis=1)
            z = jnp.dot(wa, xa_ref[:4 * xa_k, :],
                        preferred_element_type=jnp.float32)
            z = z + jnp.dot(wb, xb_ref[:4 * xa_k, :],
                            preferred_element_type=jnp.float32)
            z = z + jnp.dot(wcen, av, preferred_element_type=jnp.float32)
            ab = jnp.maximum(z + bcl, 0.0).astype(jnp.bfloat16) * mask
            if last:
                for b in range(B):
                    o_ref[0, b] = ab[:, b * SP:(b + 1) * SP]
            else:
                scr_ref[...] = ab

        layer(scr_ref[:CIN8, :], CIN8, w4a0_ref[...], w4b0_ref[...],
              wc0_ref[...], bc_ref[0], L == 1)
        for l in range(1, L):
            layer(scr_ref[...], C, w4a_ref[l - 1], w4b_ref[l - 1],
                  wc_ref[l - 1], bc_ref[l], l + 1 == L)
    return body


def _fc_body(C, SP):
    def fc(r_ref, w_ref, b_ref, o_ref):
        acc = None
        for c in range(C):
            p = jax.lax.dot_general(
                r_ref[:, c * SP:(c + 1) * SP], w_ref[c],
                (((1,), (1,)), ((), ())),
                preferred_element_type=jnp.float32)
            acc = p if acc is None else acc + p
        o_ref[...] = acc + b_ref[...]
    return fc


def kernel(x_nchw, wc, bc, wfc, bfc, mask):
    N, cin, H, W = x_nchw.shape
    L = wc.shape[0]
    C = wc.shape[2]
    NC = bfc.shape[1]
    SP = wfc.shape[2]
    PH, PW = H + 2, W + 2
    assert SP >= PH * PW and SP % 128 == 0 and cin <= C
    assert SP - (PH - 1) * PW - (PW - 1) > PW + 1 > 0

    B = 32
    steps = -(-N // B)
    N_pad = steps * B
    SPB = B * SP

    CIN8 = min(C, ((cin + 7) // 8) * 8)
    xp = jnp.pad(x_nchw.astype(jnp.bfloat16),
                 ((0, N_pad - N), (0, CIN8 - cin), (1, 1), (1, 1)))
    xp = xp.reshape(N_pad, CIN8, PH * PW)
    xp = jnp.pad(xp, ((0, 0), (0, 0), (0, SP - PH * PW)))
    xp = xp.reshape(steps, B, CIN8, SP)
    mask_b = jnp.tile(mask, (1, B)).astype(jnp.bfloat16)

    w_all = jnp.transpose(wc, (0, 2, 1, 3))
    w4a = w_all[1:, :, [0, 1, 2, 3], :].reshape(L - 1, C, 4 * C).astype(jnp.bfloat16)
    w4b = w_all[1:, :, [5, 6, 7, 8], :].reshape(L - 1, C, 4 * C).astype(jnp.bfloat16)
    wcen = w_all[1:, :, 4, :].astype(jnp.bfloat16)
    w4a0 = w_all[0][:, [0, 1, 2, 3], :CIN8].reshape(C, 4 * CIN8).astype(jnp.bfloat16)
    w4b0 = w_all[0][:, [5, 6, 7, 8], :CIN8].reshape(C, 4 * CIN8).astype(jnp.bfloat16)
    wc0 = w_all[0][:, 4, :CIN8].astype(jnp.bfloat16)

    act = pl.pallas_call(
        _conv_body(L, C, SP, B, PW),
        out_shape=jax.ShapeDtypeStruct((steps, B, C, SP), jnp.bfloat16),
        grid=(steps,),
        in_specs=[
            pl.BlockSpec((1, B, CIN8, SP), lambda s: (s, 0, 0, 0)),
            pl.BlockSpec((1, SPB), lambda s: (0, 0)),
            pl.BlockSpec((L - 1, C, 4 * C), lambda s: (0, 0, 0)),
            pl.BlockSpec((L - 1, C, 4 * C), lambda s: (0, 0, 0)),
            pl.BlockSpec((L - 1, C, C), lambda s: (0, 0, 0)),
            pl.BlockSpec((L, C, 1), lambda s: (0, 0, 0)),
            pl.BlockSpec((C, 4 * CIN8), lambda s: (0, 0)),
            pl.BlockSpec((C, 4 * CIN8), lambda s: (0, 0)),
            pl.BlockSpec((C, CIN8), lambda s: (0, 0)),
        ],
        out_specs=pl.BlockSpec((1, B, C, SP), lambda s: (s, 0, 0, 0)),
        scratch_shapes=[
            pltpu.VMEM((C, SPB), jnp.bfloat16),
            pltpu.VMEM((4 * C, SPB), jnp.bfloat16),
            pltpu.VMEM((4 * C, SPB), jnp.bfloat16),
        ],
        compiler_params=pltpu.CompilerParams(
            dimension_semantics=("parallel",)),
    )(xp, mask_b, w4a, w4b, wcen, bc, w4a0, w4b0, wc0)

    rows = act.reshape(N_pad, C * SP)
    wfcb = wfc.astype(jnp.bfloat16)

    MB = 256 if N_pad % 256 == 0 else B
    fsteps = N_pad // MB
    scores = pl.pallas_call(
        _fc_body(C, SP),
        out_shape=jax.ShapeDtypeStruct((N_pad, NC), jnp.float32),
        grid=(fsteps,),
        in_specs=[
            pl.BlockSpec((MB, C * SP), lambda s: (s, 0)),
            pl.BlockSpec((C, NC, SP), lambda s: (0, 0, 0)),
            pl.BlockSpec((1, NC), lambda s: (0, 0)),
        ],
        out_specs=pl.BlockSpec((MB, NC), lambda s: (s, 0)),
        compiler_params=pltpu.CompilerParams(
            dimension_semantics=("parallel",)),
    )(rows, wfcb, bfc)

    return scores[:N], None, None


# speedup vs baseline: 1.3261x; 1.2255x over previous
import jax
import jax.numpy as jnp
from jax.experimental import pallas as pl
from jax.experimental.pallas import tpu as pltpu


def _conv_body(L, C, SP, B, PW):
    SPB = B * SP
    def body(x_ref, mask_ref, wa_ref, wb_ref, wc_ref, bc_ref,
             wa0_ref, wb0_ref, wc0_ref, o_ref, scr_ref, x3_ref):
        CIN8 = x_ref.shape[2]
        for b in range(B):
            scr_ref[:CIN8, b * SP:(b + 1) * SP] = x_ref[0, b]
        mask = mask_ref[...]

        def layer(av, kc, wa, wb, wcen, bcl, last):
            x3_ref[0:kc, :] = jnp.roll(av, PW, axis=1)
            x3_ref[kc:2 * kc, :] = av
            x3_ref[2 * kc:3 * kc, :] = jnp.roll(av, -PW, axis=1)
            x3 = x3_ref[:3 * kc, :]
            ym = jnp.dot(wa, x3, preferred_element_type=jnp.float32)
            yz = jnp.dot(wcen, x3, preferred_element_type=jnp.float32)
            yp = jnp.dot(wb, x3, preferred_element_type=jnp.float32)
            z = yz + jnp.roll(ym, 1, axis=1) + jnp.roll(yp, -1, axis=1)
            ab = jnp.maximum(z + bcl, 0.0).astype(jnp.bfloat16) * mask
            if last:
                for b in range(B):
                    o_ref[0, b] = ab[:, b * SP:(b + 1) * SP]
            else:
                scr_ref[...] = ab

        layer(scr_ref[:CIN8, :], CIN8, wa0_ref[...], wb0_ref[...],
              wc0_ref[...], bc_ref[0], L == 1)
        for l in range(1, L):
            layer(scr_ref[...], C, wa_ref[l - 1], wb_ref[l - 1],
                  wc_ref[l - 1], bc_ref[l], l + 1 == L)
    return body


def _fc_body(C, SP):
    def fc(r_ref, w_ref, b_ref, o_ref):
        acc = None
        for c in range(C):
            p = jax.lax.dot_general(
                r_ref[:, c * SP:(c + 1) * SP], w_ref[c],
                (((1,), (1,)), ((), ())),
                preferred_element_type=jnp.float32)
            acc = p if acc is None else acc + p
        o_ref[...] = acc + b_ref[...]
    return fc


def kernel(x_nchw, wc, bc, wfc, bfc, mask):
    N, cin, H, W = x_nchw.shape
    L = wc.shape[0]
    C = wc.shape[2]
    NC = bfc.shape[1]
    SP = wfc.shape[2]
    PH, PW = H + 2, W + 2
    assert SP >= PH * PW and SP % 128 == 0 and cin <= C
    assert SP - (PH - 1) * PW - (PW - 1) > PW + 1 > 0

    B = 32
    steps = -(-N // B)
    N_pad = steps * B
    SPB = B * SP

    CIN8 = min(C, ((cin + 7) // 8) * 8)
    xp = jnp.pad(x_nchw.astype(jnp.bfloat16),
                 ((0, N_pad - N), (0, CIN8 - cin), (1, 1), (1, 1)))
    xp = xp.reshape(N_pad, CIN8, PH * PW)
    xp = jnp.pad(xp, ((0, 0), (0, 0), (0, SP - PH * PW)))
    xp = xp.reshape(steps, B, CIN8, SP)
    mask_b = jnp.tile(mask, (1, B)).astype(jnp.bfloat16)

    w_all = jnp.transpose(wc, (0, 2, 1, 3))
    bf = jnp.bfloat16
    w3m = w_all[1:, :, [0, 3, 6], :].reshape(L - 1, C, 3 * C).astype(bf)
    w3z = w_all[1:, :, [1, 4, 7], :].reshape(L - 1, C, 3 * C).astype(bf)
    w3p = w_all[1:, :, [2, 5, 8], :].reshape(L - 1, C, 3 * C).astype(bf)
    w3m0 = w_all[0][:, [0, 3, 6], :CIN8].reshape(C, 3 * CIN8).astype(bf)
    w3z0 = w_all[0][:, [1, 4, 7], :CIN8].reshape(C, 3 * CIN8).astype(bf)
    w3p0 = w_all[0][:, [2, 5, 8], :CIN8].reshape(C, 3 * CIN8).astype(bf)

    act = pl.pallas_call(
        _conv_body(L, C, SP, B, PW),
        out_shape=jax.ShapeDtypeStruct((steps, B, C, SP), jnp.bfloat16),
        grid=(steps,),
        in_specs=[
            pl.BlockSpec((1, B, CIN8, SP), lambda s: (s, 0, 0, 0)),
            pl.BlockSpec((1, SPB), lambda s: (0, 0)),
            pl.BlockSpec((L - 1, C, 3 * C), lambda s: (0, 0, 0)),
            pl.BlockSpec((L - 1, C, 3 * C), lambda s: (0, 0, 0)),
            pl.BlockSpec((L - 1, C, 3 * C), lambda s: (0, 0, 0)),
            pl.BlockSpec((L, C, 1), lambda s: (0, 0, 0)),
            pl.BlockSpec((C, 3 * CIN8), lambda s: (0, 0)),
            pl.BlockSpec((C, 3 * CIN8), lambda s: (0, 0)),
            pl.BlockSpec((C, 3 * CIN8), lambda s: (0, 0)),
        ],
        out_specs=pl.BlockSpec((1, B, C, SP), lambda s: (s, 0, 0, 0)),
        scratch_shapes=[
            pltpu.VMEM((C, SPB), jnp.bfloat16),
            pltpu.VMEM((3 * C, SPB), jnp.bfloat16),
        ],
        compiler_params=pltpu.CompilerParams(
            dimension_semantics=("parallel",)),
    )(xp, mask_b, w3m, w3p, w3z, bc, w3m0, w3p0, w3z0)

    rows = act.reshape(N_pad, C * SP)
    wfcb = wfc.astype(jnp.bfloat16)

    MB = 256 if N_pad % 256 == 0 else B
    fsteps = N_pad // MB
    scores = pl.pallas_call(
        _fc_body(C, SP),
        out_shape=jax.ShapeDtypeStruct((N_pad, NC), jnp.float32),
        grid=(fsteps,),
        in_specs=[
            pl.BlockSpec((MB, C * SP), lambda s: (s, 0)),
            pl.BlockSpec((C, NC, SP), lambda s: (0, 0, 0)),
            pl.BlockSpec((1, NC), lambda s: (0, 0)),
        ],
        out_specs=pl.BlockSpec((MB, NC), lambda s: (s, 0)),
        compiler_params=pltpu.CompilerParams(
            dimension_semantics=("parallel",)),
    )(rows, wfcb, bfc)

    return scores[:N], None, None


# speedup vs baseline: 1.3308x; 1.0035x over previous
import jax
import jax.numpy as jnp
from jax.experimental import pallas as pl
from jax.experimental.pallas import tpu as pltpu


def _conv_body(L, C, SP, B, PW):
    SPB = B * SP
    def body(x_ref, mask_ref, wa_ref, wb_ref, wc_ref, bc_ref,
             wa0_ref, wb0_ref, wc0_ref, o_ref, x3_ref):
        CIN8 = x_ref.shape[2]
        for b in range(B):
            x3_ref[C:C + CIN8, b * SP:(b + 1) * SP] = x_ref[0, b]
        mask = mask_ref[...]

        def layer(lo, kc, wa, wb, wcen, bcl, last):
            av = x3_ref[lo + kc:lo + 2 * kc, :]
            x3_ref[lo:lo + kc, :] = jnp.roll(av, PW, axis=1)
            x3_ref[lo + 2 * kc:lo + 3 * kc, :] = jnp.roll(av, -PW, axis=1)
            x3 = x3_ref[lo:lo + 3 * kc, :]
            ym = jnp.dot(wa, x3, preferred_element_type=jnp.float32)
            yz = jnp.dot(wcen, x3, preferred_element_type=jnp.float32)
            yp = jnp.dot(wb, x3, preferred_element_type=jnp.float32)
            z = yz + jnp.roll(ym, 1, axis=1) + jnp.roll(yp, -1, axis=1)
            ab = jnp.maximum(z + bcl, 0.0).astype(jnp.bfloat16) * mask
            if last:
                for b in range(B):
                    o_ref[0, b] = ab[:, b * SP:(b + 1) * SP]
            else:
                x3_ref[C:2 * C, :] = ab

        layer(C - CIN8, CIN8, wa0_ref[...], wb0_ref[...],
              wc0_ref[...], bc_ref[0], L == 1)
        for l in range(1, L):
            layer(0, C, wa_ref[l - 1], wb_ref[l - 1],
                  wc_ref[l - 1], bc_ref[l], l + 1 == L)
    return body


def _fc_body(C, SP):
    def fc(r_ref, w_ref, b_ref, o_ref):
        acc = None
        for c in range(C):
            p = jax.lax.dot_general(
                r_ref[:, c * SP:(c + 1) * SP], w_ref[c],
                (((1,), (1,)), ((), ())),
                preferred_element_type=jnp.float32)
            acc = p if acc is None else acc + p
        o_ref[...] = acc + b_ref[...]
    return fc


def kernel(x_nchw, wc, bc, wfc, bfc, mask):
    N, cin, H, W = x_nchw.shape
    L = wc.shape[0]
    C = wc.shape[2]
    NC = bfc.shape[1]
    SP = wfc.shape[2]
    PH, PW = H + 2, W + 2
    assert SP >= PH * PW and SP % 128 == 0 and cin <= C
    assert SP - (PH - 1) * PW - (PW - 1) > PW + 1 > 0

    B = 32
    steps = -(-N // B)
    N_pad = steps * B
    SPB = B * SP

    CIN8 = min(C, ((cin + 7) // 8) * 8)
    xp = jnp.pad(x_nchw.astype(jnp.bfloat16),
                 ((0, N_pad - N), (0, CIN8 - cin), (1, 1), (1, 1)))
    xp = xp.reshape(N_pad, CIN8, PH * PW)
    xp = jnp.pad(xp, ((0, 0), (0, 0), (0, SP - PH * PW)))
    xp = xp.reshape(steps, B, CIN8, SP)
    mask_b = jnp.tile(mask, (1, B)).astype(jnp.bfloat16)

    w_all = jnp.transpose(wc, (0, 2, 1, 3))
    bf = jnp.bfloat16
    w3m = w_all[1:, :, [0, 3, 6], :].reshape(L - 1, C, 3 * C).astype(bf)
    w3z = w_all[1:, :, [1, 4, 7], :].reshape(L - 1, C, 3 * C).astype(bf)
    w3p = w_all[1:, :, [2, 5, 8], :].reshape(L - 1, C, 3 * C).astype(bf)
    w3m0 = w_all[0][:, [0, 3, 6], :CIN8].reshape(C, 3 * CIN8).astype(bf)
    w3z0 = w_all[0][:, [1, 4, 7], :CIN8].reshape(C, 3 * CIN8).astype(bf)
    w3p0 = w_all[0][:, [2, 5, 8], :CIN8].reshape(C, 3 * CIN8).astype(bf)

    act = pl.pallas_call(
        _conv_body(L, C, SP, B, PW),
        out_shape=jax.ShapeDtypeStruct((steps, B, C, SP), jnp.bfloat16),
        grid=(steps,),
        in_specs=[
            pl.BlockSpec((1, B, CIN8, SP), lambda s: (s, 0, 0, 0)),
            pl.BlockSpec((1, SPB), lambda s: (0, 0)),
            pl.BlockSpec((L - 1, C, 3 * C), lambda s: (0, 0, 0)),
            pl.BlockSpec((L - 1, C, 3 * C), lambda s: (0, 0, 0)),
            pl.BlockSpec((L - 1, C, 3 * C), lambda s: (0, 0, 0)),
            pl.BlockSpec((L, C, 1), lambda s: (0, 0, 0)),
            pl.BlockSpec((C, 3 * CIN8), lambda s: (0, 0)),
            pl.BlockSpec((C, 3 * CIN8), lambda s: (0, 0)),
            pl.BlockSpec((C, 3 * CIN8), lambda s: (0, 0)),
        ],
        out_specs=pl.BlockSpec((1, B, C, SP), lambda s: (s, 0, 0, 0)),
        scratch_shapes=[
            pltpu.VMEM((3 * C, SPB), jnp.bfloat16),
        ],
        compiler_params=pltpu.CompilerParams(
            dimension_semantics=("parallel",)),
    )(xp, mask_b, w3m, w3p, w3z, bc, w3m0, w3p0, w3z0)

    rows = act.reshape(N_pad, C * SP)
    wfcb = wfc.astype(jnp.bfloat16)

    MB = 256 if N_pad % 256 == 0 else B
    fsteps = N_pad // MB
    scores = pl.pallas_call(
        _fc_body(C, SP),
        out_shape=jax.ShapeDtypeStruct((N_pad, NC), jnp.float32),
        grid=(fsteps,),
        in_specs=[
            pl.BlockSpec((MB, C * SP), lambda s: (s, 0)),
            pl.BlockSpec((C, NC, SP), lambda s: (0, 0, 0)),
            pl.BlockSpec((1, NC), lambda s: (0, 0)),
        ],
        out_specs=pl.BlockSpec((MB, NC), lambda s: (s, 0)),
        compiler_params=pltpu.CompilerParams(
            dimension_semantics=("parallel",)),
    )(rows, wfcb, bfc)

    return scores[:N], None, None
